# baseline (speedup 1.0000x reference)
import jax
import jax.numpy as jnp
from jax import lax
from jax.experimental import pallas as pl
from jax.experimental.pallas import tpu as pltpu

_VOCAB = 100000
_EMBED = 64
_CTX = 200
_HIDDEN = 128

_CB = 8192
_NFULL = _VOCAB // _CB
_TAIL = _VOCAB - _NFULL * _CB
_PADV = (_NFULL + 1) * _CB
_NBUF = 8
_NWIN = 200


def _body(idx_ref, w1_ref, b1_ref, b2_ref, tab_hbm, w2t_hbm,
          out_ref, emb_ref, wins, bufs, tail_buf, out_s, acc_s,
          win_sems, sems, tail_sem):
    def chunk_copy(c, slot):
        return pltpu.make_async_copy(
            w2t_hbm.at[pl.ds(c * _CB, _CB), :], bufs.at[slot], sems.at[slot]
        )

    for k in range(_NBUF):
        chunk_copy(k, k).start()
    tail_cp = pltpu.make_async_copy(
        w2t_hbm.at[pl.ds(_NFULL * _CB, _TAIL), :], tail_buf, tail_sem
    )
    tail_cp.start()

    def win_copy(r, slot):
        i = idx_ref[r]
        c0 = pl.multiple_of((i // 128) * 128, 128)
        return pltpu.make_async_copy(
            tab_hbm.at[:, pl.ds(c0, 128)], wins.at[slot], win_sems.at[slot]
        )

    for r in range(_NWIN):
        win_copy(r, r).start()

    acc_s[...] = jnp.zeros((_EMBED, 128), jnp.float32)
    lane = lax.broadcasted_iota(jnp.int32, (_EMBED, 128), 1)
    for r in range(_CTX):
        slot = r % _NWIN
        win_copy(r, slot).wait()
        i = idx_ref[r]
        acc_s[...] += jnp.where(lane == (i % 128), wins[slot], 0.0)
        nxt = r + _NWIN
        if nxt < _CTX:
            win_copy(nxt, slot).start()

    embeds = jnp.sum(acc_s[...], axis=1)
    emb_ref[...] = embeds

    h = jnp.maximum(
        jax.lax.dot_general(
            embeds.reshape(1, _EMBED), w1_ref[...], (((1,), (0,)), ((), ())),
            preferred_element_type=jnp.float32,
        ) + b1_ref[...],
        0.0,
    )

    for c in range(_NFULL):
        slot = c % _NBUF
        chunk_copy(c, slot).wait()
        blk = jax.lax.dot_general(
            h, bufs[slot], (((1,), (1,)), ((), ())),
            preferred_element_type=jnp.float32,
        ) + b2_ref[:, pl.ds(c * _CB, _CB)]
        out_s[:, pl.ds(c * _CB, _CB)] = blk
        nxt = c + _NBUF
        if nxt < _NFULL:
            chunk_copy(nxt, slot).start()

    tail_cp.wait()
    blk_t = jax.lax.dot_general(
        h, tail_buf[...], (((1,), (1,)), ((), ())),
        preferred_element_type=jnp.float32,
    ) + b2_ref[:, pl.ds(_NFULL * _CB, _TAIL)]
    out_s[:, pl.ds(_NFULL * _CB, _TAIL)] = blk_t

    full = out_s[...]
    col = lax.broadcasted_iota(jnp.int32, (1, _PADV), 1)
    valid = col < _VOCAB
    m = jnp.max(jnp.where(valid, full, -jnp.inf))
    e = jnp.where(valid, jnp.exp(full - m), 0.0)
    lse = m + jnp.log(jnp.sum(e))
    out_ref[...] = (full - lse)[:, :_VOCAB]


def _fused(idx, tab_t, W1, b1, w2t, b2):
    return pl.pallas_call(
        _body,
        in_specs=[
            pl.BlockSpec(memory_space=pltpu.SMEM),
            pl.BlockSpec((_EMBED, _HIDDEN), lambda: (0, 0)),
            pl.BlockSpec((1, _HIDDEN), lambda: (0, 0)),
            pl.BlockSpec((1, _VOCAB), lambda: (0, 0)),
            pl.BlockSpec(memory_space=pl.ANY),
            pl.BlockSpec(memory_space=pl.ANY),
        ],
        out_specs=[
            pl.BlockSpec((1, _VOCAB), lambda: (0, 0)),
            pl.BlockSpec(memory_space=pltpu.VMEM),
        ],
        out_shape=[
            jax.ShapeDtypeStruct((1, _VOCAB), jnp.float32),
            jax.ShapeDtypeStruct((_EMBED,), jnp.float32),
        ],
        scratch_shapes=[
            pltpu.VMEM((_NWIN, _EMBED, 128), jnp.float32),
            pltpu.VMEM((_NBUF, _CB, _HIDDEN), jnp.float32),
            pltpu.VMEM((_TAIL, _HIDDEN), jnp.float32),
            pltpu.VMEM((1, _PADV), jnp.float32),
            pltpu.VMEM((_EMBED, 128), jnp.float32),
            pltpu.SemaphoreType.DMA((_NWIN,)),
            pltpu.SemaphoreType.DMA((_NBUF,)),
            pltpu.SemaphoreType.DMA,
        ],
    )(idx, W1, b1.reshape(1, _HIDDEN), b2.reshape(1, _VOCAB), tab_t, w2t)


def kernel(inputs, emb_table, W1, b1, W2, b2):
    idx = inputs.astype(jnp.int32)
    log_probs, embeds = _fused(idx, emb_table.T, W1, b1, W2.T, b2)
    return (log_probs, embeds)


# speedup vs baseline: 1.0065x; 1.0065x over previous
import jax
import jax.numpy as jnp
from jax import lax
from jax.experimental import pallas as pl
from jax.experimental.pallas import tpu as pltpu

_VOCAB = 100000
_EMBED = 64
_CTX = 200
_HIDDEN = 128

_CB = 8192
_NFULL = _VOCAB // _CB
_TAIL = _VOCAB - _NFULL * _CB
_PADV = (_NFULL + 1) * _CB
_NBUF = 8
_NWIN = 64


def _body(idx_ref, w1_ref, b1_ref, b2_ref, tab_hbm, w2t_hbm,
          out_ref, emb_ref, wins, bufs, tail_buf, out_s, acc_s,
          win_sems, sems, tail_sem):
    def chunk_copy(c, slot):
        return pltpu.make_async_copy(
            w2t_hbm.at[pl.ds(c * _CB, _CB), :], bufs.at[slot], sems.at[slot]
        )

    for k in range(_NBUF):
        chunk_copy(k, k).start()
    tail_cp = pltpu.make_async_copy(
        w2t_hbm.at[pl.ds(_NFULL * _CB, _TAIL), :], tail_buf, tail_sem
    )
    tail_cp.start()

    def win_copy(r, slot):
        i = idx_ref[r]
        c0 = pl.multiple_of((i // 128) * 128, 128)
        return pltpu.make_async_copy(
            tab_hbm.at[:, pl.ds(c0, 128)], wins.at[slot], win_sems.at[slot]
        )

    for r in range(_NWIN):
        win_copy(r, r).start()

    acc_s[...] = jnp.zeros((_EMBED, 128), jnp.float32)
    lane = lax.broadcasted_iota(jnp.int32, (_EMBED, 128), 1)
    for r in range(_CTX):
        slot = r % _NWIN
        win_copy(r, slot).wait()
        i = idx_ref[r]
        acc_s[...] += jnp.where(lane == (i % 128), wins[slot], 0.0)
        nxt = r + _NWIN
        if nxt < _CTX:
            win_copy(nxt, slot).start()

    embeds = jnp.sum(acc_s[...], axis=1)
    emb_ref[...] = embeds

    h = jnp.maximum(
        jax.lax.dot_general(
            embeds.reshape(1, _EMBED), w1_ref[...], (((1,), (0,)), ((), ())),
            preferred_element_type=jnp.float32,
        ) + b1_ref[...],
        0.0,
    )

    for c in range(_NFULL):
        slot = c % _NBUF
        chunk_copy(c, slot).wait()
        blk = jax.lax.dot_general(
            h, bufs[slot], (((1,), (1,)), ((), ())),
            preferred_element_type=jnp.float32,
        ) + b2_ref[:, pl.ds(c * _CB, _CB)]
        out_s[:, pl.ds(c * _CB, _CB)] = blk
        nxt = c + _NBUF
        if nxt < _NFULL:
            chunk_copy(nxt, slot).start()

    tail_cp.wait()
    blk_t = jax.lax.dot_general(
        h, tail_buf[...], (((1,), (1,)), ((), ())),
        preferred_element_type=jnp.float32,
    ) + b2_ref[:, pl.ds(_NFULL * _CB, _TAIL)]
    out_s[:, pl.ds(_NFULL * _CB, _TAIL)] = blk_t

    full = out_s[...]
    col = lax.broadcasted_iota(jnp.int32, (1, _PADV), 1)
    valid = col < _VOCAB
    m = jnp.max(jnp.where(valid, full, -jnp.inf))
    e = jnp.where(valid, jnp.exp(full - m), 0.0)
    lse = m + jnp.log(jnp.sum(e))
    out_ref[...] = (full - lse)[:, :_VOCAB]


def _fused(idx, tab_t, W1, b1, w2t, b2):
    return pl.pallas_call(
        _body,
        in_specs=[
            pl.BlockSpec(memory_space=pltpu.SMEM),
            pl.BlockSpec((_EMBED, _HIDDEN), lambda: (0, 0)),
            pl.BlockSpec((1, _HIDDEN), lambda: (0, 0)),
            pl.BlockSpec((1, _VOCAB), lambda: (0, 0)),
            pl.BlockSpec(memory_space=pl.ANY),
            pl.BlockSpec(memory_space=pl.ANY),
        ],
        out_specs=[
            pl.BlockSpec((1, _VOCAB), lambda: (0, 0)),
            pl.BlockSpec(memory_space=pltpu.VMEM),
        ],
        out_shape=[
            jax.ShapeDtypeStruct((1, _VOCAB), jnp.float32),
            jax.ShapeDtypeStruct((_EMBED,), jnp.float32),
        ],
        scratch_shapes=[
            pltpu.VMEM((_NWIN, _EMBED, 128), jnp.float32),
            pltpu.VMEM((_NBUF, _CB, _HIDDEN), jnp.float32),
            pltpu.VMEM((_TAIL, _HIDDEN), jnp.float32),
            pltpu.VMEM((1, _PADV), jnp.float32),
            pltpu.VMEM((_EMBED, 128), jnp.float32),
            pltpu.SemaphoreType.DMA((_NWIN,)),
            pltpu.SemaphoreType.DMA((_NBUF,)),
            pltpu.SemaphoreType.DMA,
        ],
    )(idx, W1, b1.reshape(1, _HIDDEN), b2.reshape(1, _VOCAB), tab_t, w2t)


def kernel(inputs, emb_table, W1, b1, W2, b2):
    idx = inputs.astype(jnp.int32)
    log_probs, embeds = _fused(idx, emb_table.T, W1, b1, W2.T, b2)
    return (log_probs, embeds)
